# baseline (speedup 1.0000x reference)
import jax
import jax.numpy as jnp
from jax import lax
from jax.experimental import pallas as pl
from jax.experimental.pallas import tpu as pltpu
from jax.experimental.pallas import tpu_sc as plsc

N_NODES = 10000
N_EDGES = 320000
F = 128
L = 16

NC = 2
NS = 16
NW = NC * NS

CHUNK = 128
EPT = N_EDGES // NW
NCHUNK = 80
EPT_PAD = NCHUNK * CHUNK
ACC_ROWS = 10240
TROWS = ACC_ROWS // NS


def _sc_body(x_hbm, packed_hbm, zeros_hbm, out_hbm,
             packed_v, srcc0, srcc1, dstc0, dstc1, rows0, rows1,
             acc_sh, gs0, gs1):
    c = lax.axis_index("c")
    s = lax.axis_index("s")
    wid = c * NS + s
    pltpu.sync_copy(packed_hbm.at[wid], packed_v)
    pltpu.sync_copy(zeros_hbm.at[pl.ds(s * TROWS, TROWS)],
                    acc_sh.at[pl.ds(s * TROWS, TROWS)])
    plsc.subcore_barrier()

    def unpack(jj, src_c, dst_c):
        for k in range(CHUNK // L):
            p = packed_v[jj, pl.ds(k * L, L)]
            src_c[0, pl.ds(k * L, L)] = p & 0xFFFF
            dst_c[0, pl.ds(k * L, L)] = p >> 16

    unpack(0, srcc0, dstc0)
    unpack(1, srcc1, dstc1)
    pltpu.async_copy(x_hbm.at[srcc0.at[0]], rows0, gs0)
    pltpu.async_copy(x_hbm.at[srcc1.at[0]], rows1, gs1)

    def step(i, carry):
        j = 2 * i
        pltpu.make_async_copy(x_hbm.at[srcc0.at[0]], rows0, gs0).wait()
        pltpu.sync_copy(rows0, acc_sh.at[dstc0.at[0]], add=True)

        @pl.when(j + 2 < NCHUNK)
        def _():
            unpack(j + 2, srcc0, dstc0)
            pltpu.async_copy(x_hbm.at[srcc0.at[0]], rows0, gs0)

        pltpu.make_async_copy(x_hbm.at[srcc1.at[0]], rows1, gs1).wait()
        pltpu.sync_copy(rows1, acc_sh.at[dstc1.at[0]], add=True)

        @pl.when(j + 3 < NCHUNK)
        def _():
            unpack(j + 3, srcc1, dstc1)
            pltpu.async_copy(x_hbm.at[srcc1.at[0]], rows1, gs1)

        return carry

    lax.fori_loop(0, NCHUNK // 2, step, 0)
    plsc.subcore_barrier()
    pltpu.sync_copy(acc_sh.at[pl.ds(s * TROWS, TROWS)],
                    out_hbm.at[c, pl.ds(s * TROWS, TROWS)])


@jax.jit
def _sc_scatter(x, packed, zeros):
    mesh = plsc.VectorSubcoreMesh(core_axis_name="c", subcore_axis_name="s",
                                  num_cores=NC, num_subcores=NS)
    return pl.kernel(
        _sc_body,
        out_type=jax.ShapeDtypeStruct((NC, ACC_ROWS, F), jnp.float32),
        mesh=mesh,
        scratch_types=[
            pltpu.VMEM((NCHUNK, CHUNK), jnp.int32),
            pltpu.VMEM((1, CHUNK), jnp.int32),
            pltpu.VMEM((1, CHUNK), jnp.int32),
            pltpu.VMEM((1, CHUNK), jnp.int32),
            pltpu.VMEM((1, CHUNK), jnp.int32),
            pltpu.VMEM((CHUNK, F), jnp.float32),
            pltpu.VMEM((CHUNK, F), jnp.float32),
            pltpu.VMEM_SHARED((ACC_ROWS, F), jnp.float32),
            pltpu.SemaphoreType.DMA,
            pltpu.SemaphoreType.DMA,
        ],
    )(x, packed, zeros)


def _tc_body(p_ref, w_ref, b_ref, g_ref, be_ref, out_ref):
    h = p_ref[0, :N_NODES] + p_ref[1, :N_NODES]
    y = lax.dot_general(h, w_ref[...], (((1,), (1,)), ((), ())),
                        preferred_element_type=jnp.float32,
                        precision=lax.Precision.HIGHEST)
    y = jnp.maximum(y + b_ref[...], 0.0)
    mean = jnp.mean(y, axis=0, keepdims=True)
    var = jnp.mean(jnp.square(y - mean), axis=0, keepdims=True)
    out_ref[...] = (y - mean) * lax.rsqrt(var + 1e-5) * g_ref[...] + be_ref[...]


@jax.jit
def _tc_finish(partials, W, b, gamma, beta):
    return pl.pallas_call(
        _tc_body,
        out_shape=jax.ShapeDtypeStruct((N_NODES, F), jnp.float32),
    )(partials, W, b.reshape(1, F), gamma.reshape(1, F), beta.reshape(1, F))


def kernel(x, edge_index, W, b, gamma, beta):
    src = edge_index[0].astype(jnp.int32).reshape(NW, EPT)
    dst = edge_index[1].astype(jnp.int32).reshape(NW, EPT)
    pad = EPT_PAD - EPT
    src_p = jnp.pad(src, ((0, 0), (0, pad)))
    dst_p = jnp.pad(dst, ((0, 0), (0, pad)), constant_values=N_NODES)
    packed = (src_p | (dst_p << 16)).reshape(NW, NCHUNK, CHUNK)
    zeros = jnp.zeros((ACC_ROWS, F), jnp.float32)
    partials = _sc_scatter(x, packed, zeros)
    return _tc_finish(partials, W, b, gamma, beta)
